# baseline (speedup 1.0000x reference)
import functools

import jax
import jax.numpy as jnp
from jax import lax
from jax.experimental import pallas as pl
from jax.experimental.pallas import tpu as pltpu
from jax.experimental.pallas import tpu_sc as plsc

_NC = 2
_NS = 16
_NW = _NC * _NS
_C = 64


def _gather_sum(table, idx2d, npad):
    d = table.shape[1]
    s = (idx2d.shape[0] * idx2d.shape[1]) // npad
    cp = 128 // s
    rpc = (cp * s) // 128
    npw = npad // _NW
    nch = npw // cp
    grp = 4
    ngr = nch // grp
    nk = d // 16
    mesh = plsc.VectorSubcoreMesh(core_axis_name="c", subcore_axis_name="s")

    @functools.partial(
        pl.kernel,
        out_type=jax.ShapeDtypeStruct((npad, d), jnp.float32),
        mesh=mesh,
        scratch_types=[
            pltpu.VMEM((nch * rpc, 128), jnp.int32),
            pltpu.VMEM((2, cp * s, d), jnp.float32),
            pltpu.VMEM((2, grp * cp, d), jnp.float32),
            pltpu.VMEM_SHARED(table.shape, jnp.float32),
            [pltpu.SemaphoreType.DMA] * 2,
            [pltpu.SemaphoreType.DMA] * 2,
        ],
    )
    def gather_kernel(table_hbm, idx_hbm, out_hbm, idxb, buf, ostage,
                      shared_t, semG, semO):
        cid = lax.axis_index("c")
        sid = lax.axis_index("s")
        t0 = cid * (_NS * nch) + sid * nch
        node0 = t0 * cp

        pltpu.sync_copy(idx_hbm.at[pl.ds(t0 * rpc, nch * rpc)], idxb)

        @pl.when(sid == 0)
        def _():
            pltpu.sync_copy(table_hbm, shared_t)

        plsc.subcore_barrier()

        def fire(q):
            p = q % 2
            for half in range(rpc):
                pltpu.async_copy(shared_t.at[idxb.at[q * rpc + half]],
                                 buf.at[p].at[pl.ds(half * 128, 128)],
                                 semG[p])

        fire(0)
        for g in range(ngr):
            fb = g % 2
            if g >= 2:
                pltpu.make_async_copy(
                    ostage.at[fb],
                    out_hbm.at[pl.ds(node0 + (g - 2) * grp * cp, grp * cp)],
                    semO[fb]).wait()
            for qq in range(grp):
                q = g * grp + qq
                p = q % 2
                if q + 1 < nch:
                    fire(q + 1)
                with jax.named_scope("gwait"):
                    for half in range(rpc):
                        pltpu.make_async_copy(
                            table_hbm.at[idxb.at[0]],
                            buf.at[p].at[pl.ds(half * 128, 128)],
                            semG[p]).wait()

                def node_sum(i, cy):
                    accs = [jnp.zeros((16,), jnp.float32)] * nk

                    def rows(r, accs):
                        base = i * s + r
                        return tuple(
                            accs[k] + buf[p, base, pl.ds(k * 16, 16)]
                            for k in range(nk))

                    accs = lax.fori_loop(0, s, rows, tuple(accs), unroll=4)
                    for k in range(nk):
                        ostage[fb, qq * cp + i, pl.ds(k * 16, 16)] = accs[k]
                    return cy

                with jax.named_scope("reduce"):
                    lax.fori_loop(0, cp, node_sum, 0)

            pltpu.make_async_copy(
                ostage.at[fb],
                out_hbm.at[pl.ds(node0 + g * grp * cp, grp * cp)],
                semO[fb]).start()

        for g in range(max(ngr - 2, 0), ngr):
            fb = g % 2
            pltpu.make_async_copy(
                ostage.at[fb],
                out_hbm.at[pl.ds(node0 + g * grp * cp, grp * cp)],
                semO[fb]).wait()

    return gather_kernel(table, idx2d)


_BLK = 2000


def _matmul(x, w, with_bf16=False):
    n = x.shape[0]

    def body(x_ref, w_ref, o_ref, *rest):
        r = jnp.dot(x_ref[...], w_ref[...], preferred_element_type=jnp.float32)
        o_ref[...] = r
        if rest:
            rest[0][...] = r.astype(jnp.bfloat16)

    out_specs = pl.BlockSpec((_BLK, w.shape[1]), lambda i: (i, 0))
    out_shape = jax.ShapeDtypeStruct((n, w.shape[1]), jnp.float32)
    if with_bf16:
        out_specs = (out_specs, pl.BlockSpec((_BLK, w.shape[1]),
                                             lambda i: (i, 0)))
        out_shape = (out_shape,
                     jax.ShapeDtypeStruct((n, w.shape[1]), jnp.bfloat16))
    return pl.pallas_call(
        body,
        grid=(n // _BLK,),
        in_specs=[
            pl.BlockSpec((_BLK, x.shape[1]), lambda i: (i, 0)),
            pl.BlockSpec(w.shape, lambda i: (0, 0)),
        ],
        out_specs=out_specs,
        out_shape=out_shape,
    )(x, w)


def _layer0(m_self, agg_sum, w_neigh, scale):
    n, h = m_self.shape
    d = w_neigh.shape[0]

    def body(m_ref, s_ref, wb_ref, o_ref):
        m = m_ref[...] + jnp.dot(s_ref[...] * scale, wb_ref[...],
                                 preferred_element_type=jnp.float32)
        o_ref[...] = jnp.maximum(m, 0.0)

    return pl.pallas_call(
        body,
        grid=(n // _BLK,),
        in_specs=[
            pl.BlockSpec((_BLK, h), lambda i: (i, 0)),
            pl.BlockSpec((_BLK, d), lambda i: (i, 0)),
            pl.BlockSpec(w_neigh.shape, lambda i: (0, 0)),
        ],
        out_specs=pl.BlockSpec((_BLK, h), lambda i: (i, 0)),
        out_shape=jax.ShapeDtypeStruct((n, h), jnp.float32),
    )(m_self, agg_sum, w_neigh)


def _layer1(h0, m_self, agg_sum, w_neigh, scale):
    n, h = h0.shape
    h1 = m_self.shape[1]
    d = w_neigh.shape[0]

    def body(h_ref, m_ref, s_ref, wb_ref, o_ref):
        m = m_ref[...] + jnp.dot(s_ref[...] * scale, wb_ref[...],
                                 preferred_element_type=jnp.float32)
        o_ref[0, :, :h] = h_ref[...]
        o_ref[0, :, h:] = jnp.maximum(m, 0.0)

    return pl.pallas_call(
        body,
        grid=(n // _BLK,),
        in_specs=[
            pl.BlockSpec((_BLK, h), lambda i: (i, 0)),
            pl.BlockSpec((_BLK, h1), lambda i: (i, 0)),
            pl.BlockSpec((_BLK, d), lambda i: (i, 0)),
            pl.BlockSpec(w_neigh.shape, lambda i: (0, 0)),
        ],
        out_specs=pl.BlockSpec((1, _BLK, h + h1), lambda i: (0, i, 0)),
        out_shape=jax.ShapeDtypeStruct((1, n, h + h1), jnp.float32),
    )(h0, m_self, agg_sum, w_neigh)


def kernel(feats, node_neigh, W_prep, W0, W1):
    n, s = node_neigh.shape
    p = W_prep.shape[1]
    h0_dim = W0.shape[1]
    scale = 1.0 / s

    npad = -(-n // (_NW * _C)) * (_NW * _C)
    pad_idx = (jnp.arange(s * (npad - n), dtype=jnp.int32)
               .reshape(npad - n, s)) % n
    neigh_p = jnp.concatenate([node_neigh, pad_idx], axis=0)
    idx2d = neigh_p.reshape((npad * s) // 128, 128)

    all_feats = _matmul(feats, W_prep)
    s0 = _gather_sum(all_feats, idx2d, npad)
    m0 = _matmul(all_feats, W0[:p])
    h0 = _layer0(m0, s0, W0[p:], scale)
    s1 = _gather_sum(h0, idx2d, npad)
    m1 = _matmul(h0, W1[:h0_dim])
    return _layer1(h0, m1, s1, W1[h0_dim:], scale)


# speedup vs baseline: 1.0006x; 1.0006x over previous
import functools

import jax
import jax.numpy as jnp
from jax import lax
from jax.experimental import pallas as pl
from jax.experimental.pallas import tpu as pltpu
from jax.experimental.pallas import tpu_sc as plsc

_NC = 2
_NS = 16
_NW = _NC * _NS
_C = 64


def _gather_sum(table, idx2d, npad):
    d = table.shape[1]
    s = (idx2d.shape[0] * idx2d.shape[1]) // npad
    cp = 128 // s
    rpc = (cp * s) // 128
    npw = npad // _NW
    nch = npw // cp
    grp = 4
    ngr = nch // grp
    nk = d // 16
    mesh = plsc.VectorSubcoreMesh(core_axis_name="c", subcore_axis_name="s")

    @functools.partial(
        pl.kernel,
        out_type=jax.ShapeDtypeStruct((npad, d), jnp.float32),
        mesh=mesh,
        scratch_types=[
            pltpu.VMEM((nch * rpc, 128), jnp.int32),
            pltpu.VMEM((2, cp * s, d), jnp.float32),
            pltpu.VMEM((2, grp * cp, d), jnp.float32),
            pltpu.VMEM_SHARED(table.shape, jnp.float32),
            [pltpu.SemaphoreType.DMA] * 2,
            [pltpu.SemaphoreType.DMA] * 2,
        ],
    )
    def gather_kernel(table_hbm, idx_hbm, out_hbm, idxb, buf, ostage,
                      shared_t, semG, semO):
        cid = lax.axis_index("c")
        sid = lax.axis_index("s")
        t0 = cid * (_NS * nch) + sid * nch
        node0 = t0 * cp

        pltpu.sync_copy(idx_hbm.at[pl.ds(t0 * rpc, nch * rpc)], idxb)

        @pl.when(sid == 0)
        def _():
            pltpu.sync_copy(table_hbm, shared_t)

        plsc.subcore_barrier()

        def fire(q):
            p = q % 2
            for half in range(rpc):
                pltpu.async_copy(shared_t.at[idxb.at[q * rpc + half]],
                                 buf.at[p].at[pl.ds(half * 128, 128)],
                                 semG[p])

        fire(0)
        for g in range(ngr):
            fb = g % 2
            if g >= 2:
                pltpu.make_async_copy(
                    ostage.at[fb],
                    out_hbm.at[pl.ds(node0 + (g - 2) * grp * cp, grp * cp)],
                    semO[fb]).wait()
            for qq in range(grp):
                q = g * grp + qq
                p = q % 2
                if q + 1 < nch:
                    fire(q + 1)
                with jax.named_scope("gwait"):
                    for half in range(rpc):
                        pltpu.make_async_copy(
                            table_hbm.at[idxb.at[0]],
                            buf.at[p].at[pl.ds(half * 128, 128)],
                            semG[p]).wait()

                def node_sum(i, cy):
                    accs = [jnp.zeros((16,), jnp.float32)] * nk

                    def rows(r, accs):
                        base = i * s + r
                        return tuple(
                            accs[k] + buf[p, base, pl.ds(k * 16, 16)]
                            for k in range(nk))

                    accs = lax.fori_loop(0, s, rows, tuple(accs), unroll=4)
                    for k in range(nk):
                        ostage[fb, qq * cp + i, pl.ds(k * 16, 16)] = accs[k]
                    return cy

                with jax.named_scope("reduce"):
                    lax.fori_loop(0, cp, node_sum, 0)

            pltpu.make_async_copy(
                ostage.at[fb],
                out_hbm.at[pl.ds(node0 + g * grp * cp, grp * cp)],
                semO[fb]).start()

        for g in range(max(ngr - 2, 0), ngr):
            fb = g % 2
            pltpu.make_async_copy(
                ostage.at[fb],
                out_hbm.at[pl.ds(node0 + g * grp * cp, grp * cp)],
                semO[fb]).wait()

    return gather_kernel(table, idx2d)


_BLK = 2000


def _matmul(x, w):
    n = x.shape[0]

    def body(x_ref, w_ref, o_ref):
        o_ref[...] = jnp.dot(x_ref[...], w_ref[...],
                             preferred_element_type=jnp.float32)

    return pl.pallas_call(
        body,
        grid=(n // _BLK,),
        in_specs=[
            pl.BlockSpec((_BLK, x.shape[1]), lambda i: (i, 0)),
            pl.BlockSpec(w.shape, lambda i: (0, 0)),
        ],
        out_specs=pl.BlockSpec((_BLK, w.shape[1]), lambda i: (i, 0)),
        out_shape=jax.ShapeDtypeStruct((n, w.shape[1]), jnp.float32),
    )(x, w)


def _layer0(m_self, agg_sum, w_neigh, scale):
    n, h = m_self.shape
    d = w_neigh.shape[0]

    def body(m_ref, s_ref, wb_ref, o_ref):
        m = m_ref[...] + jnp.dot(s_ref[...] * scale, wb_ref[...],
                                 preferred_element_type=jnp.float32)
        o_ref[...] = jnp.maximum(m, 0.0)

    return pl.pallas_call(
        body,
        grid=(n // _BLK,),
        in_specs=[
            pl.BlockSpec((_BLK, h), lambda i: (i, 0)),
            pl.BlockSpec((_BLK, d), lambda i: (i, 0)),
            pl.BlockSpec(w_neigh.shape, lambda i: (0, 0)),
        ],
        out_specs=pl.BlockSpec((_BLK, h), lambda i: (i, 0)),
        out_shape=jax.ShapeDtypeStruct((n, h), jnp.float32),
    )(m_self, agg_sum, w_neigh)


def _layer1(h0, m_self, agg_sum, w_neigh, scale):
    n, h = h0.shape
    h1 = m_self.shape[1]
    d = w_neigh.shape[0]

    def body(h_ref, m_ref, s_ref, wb_ref, o_ref):
        m = m_ref[...] + jnp.dot(s_ref[...] * scale, wb_ref[...],
                                 preferred_element_type=jnp.float32)
        o_ref[0, :, :h] = h_ref[...]
        o_ref[0, :, h:] = jnp.maximum(m, 0.0)

    return pl.pallas_call(
        body,
        grid=(n // _BLK,),
        in_specs=[
            pl.BlockSpec((_BLK, h), lambda i: (i, 0)),
            pl.BlockSpec((_BLK, h1), lambda i: (i, 0)),
            pl.BlockSpec((_BLK, d), lambda i: (i, 0)),
            pl.BlockSpec(w_neigh.shape, lambda i: (0, 0)),
        ],
        out_specs=pl.BlockSpec((1, _BLK, h + h1), lambda i: (0, i, 0)),
        out_shape=jax.ShapeDtypeStruct((1, n, h + h1), jnp.float32),
    )(h0, m_self, agg_sum, w_neigh)


def kernel(feats, node_neigh, W_prep, W0, W1):
    n, s = node_neigh.shape
    p = W_prep.shape[1]
    h0_dim = W0.shape[1]
    scale = 1.0 / s

    npad = -(-n // (_NW * _C)) * (_NW * _C)
    pad_idx = (jnp.arange(s * (npad - n), dtype=jnp.int32)
               .reshape(npad - n, s)) % n
    neigh_p = jnp.concatenate([node_neigh, pad_idx], axis=0)
    idx2d = neigh_p.reshape((npad * s) // 128, 128)

    all_feats = _matmul(feats, W_prep)
    s0 = _gather_sum(all_feats, idx2d, npad)
    m0 = _matmul(all_feats, W0[:p])
    h0 = _layer0(m0, s0, W0[p:], scale)
    s1 = _gather_sum(h0, idx2d, npad)
    m1 = _matmul(h0, W1[:h0_dim])
    return _layer1(h0, m1, s1, W1[h0_dim:], scale)
